# Initial kernel scaffold; baseline (speedup 1.0000x reference)
import jax
import jax.numpy as jnp
from jax.experimental import pallas as pl


def kernel(inputs, inputs_s, inputs_s1, indexes, k, features, labels, pred_labels):
    raise NotImplementedError("write your pallas kernel here")


# speedup vs baseline: 2.9540x; 2.9540x over previous
import functools

import jax
import jax.numpy as jnp
from jax.experimental import pallas as pl

TEMP = 0.05
CLS = 1000
L = 2 * CLS
NEG = -1e30


def _bank_kernel(m_total, tile,
                 x_ref, f_ref, labr_ref, labc_ref, plabr_ref,
                 idxr_ref, idxc_ref,
                 cls_ref, cnt_ref, tsum_ref, ssum_ref, tcnt_ref, scnt_ref,
                 topk_ref, tgt_ref, blab_ref):
    i = pl.program_id(0)
    first = (i == 0)
    x = x_ref[...]
    f = f_ref[...]
    labr = labr_ref[0]
    labc = labc_ref[...]
    plabr = plabr_ref[0]
    B = x.shape[0]
    T = f.shape[0]

    il = jax.lax.broadcasted_iota(jnp.int32, (L, T), 0)
    oh = (il == labr).astype(jnp.float32)
    csum = jnp.dot(oh, f, preferred_element_type=jnp.float32)
    ccnt = jnp.sum(oh, axis=1, keepdims=True)
    cls_ref[...] = jnp.where(first, 0.0, cls_ref[...]) + csum
    cnt_ref[...] = (jnp.where(first, 0.0, cnt_ref[...])
                    + jnp.broadcast_to(ccnt, cnt_ref.shape))

    ic = jax.lax.broadcasted_iota(jnp.int32, (CLS, T), 0)
    pon = (ic == plabr).astype(jnp.float32)
    tmask = (labr >= CLS).astype(jnp.float32)
    smask = jnp.logical_and(labr >= 0, labr < CLS).astype(jnp.float32)
    toh = pon * tmask
    soh = pon * smask
    tsum_ref[...] = (jnp.where(first, 0.0, tsum_ref[...])
                     + jnp.dot(toh, f, preferred_element_type=jnp.float32))
    ssum_ref[...] = (jnp.where(first, 0.0, ssum_ref[...])
                     + jnp.dot(soh, f, preferred_element_type=jnp.float32))
    tcnt_ref[...] = (jnp.where(first, 0.0, tcnt_ref[...])
                     + jnp.broadcast_to(jnp.sum(toh, axis=1, keepdims=True),
                                        tcnt_ref.shape))
    scnt_ref[...] = (jnp.where(first, 0.0, scnt_ref[...])
                     + jnp.broadcast_to(jnp.sum(soh, axis=1, keepdims=True),
                                        scnt_ref.shape))

    sim = jnp.dot(x, f.T, preferred_element_type=jnp.float32) / TEMP
    gir = jax.lax.broadcasted_iota(jnp.int32, (1, T), 1) + i * T
    sim = jnp.where(gir < m_total, sim, NEG)

    work = sim
    ms = []
    big = jnp.int32(2 ** 30)
    for _ in range(10):
        m = jnp.max(work, axis=1, keepdims=True)
        hit = jnp.where(work == m, gir, big)
        fidx = jnp.min(hit, axis=1, keepdims=True)
        ms.append(m)
        work = jnp.where(gir == fidx, NEG, work)
    tile_top = jnp.concatenate(ms, axis=1)
    cur = jnp.where(first, NEG, topk_ref[...])[:, :10]
    cand = jnp.concatenate([cur, tile_top], axis=1)
    i20 = jax.lax.broadcasted_iota(jnp.int32, (1, 20), 1)
    ms2 = []
    for _ in range(10):
        m = jnp.max(cand, axis=1, keepdims=True)
        hit = jnp.where(cand == m, i20, big)
        fidx = jnp.min(hit, axis=1, keepdims=True)
        ms2.append(m)
        cand = jnp.where(i20 == fidx, NEG, cand)
    newtop = jnp.concatenate(ms2, axis=1)
    topk_ref[...] = jnp.concatenate(
        [newtop, jnp.full((B, topk_ref.shape[1] - 10), NEG, jnp.float32)],
        axis=1)

    idxr = idxr_ref[...]
    idxc = idxc_ref[...]
    gic = jax.lax.broadcasted_iota(jnp.int32, (T, 1), 0) + i * T
    match_tb = (gic == idxr)
    tgt_c = jnp.sum(jnp.where(match_tb, labc.astype(jnp.float32), 0.0),
                    axis=0, keepdims=True)
    tgt_ref[...] = jnp.where(first, 0.0, tgt_ref[...]) + tgt_c
    match_bt = (idxc == gir)
    blab_c = jnp.sum(jnp.where(match_bt, plabr.astype(jnp.float32), 0.0),
                     axis=1, keepdims=True)
    blab_ref[...] = jnp.where(first, 0.0, blab_ref[...]) + blab_c


def _finish_kernel(x_ref, xs_ref, xs1_ref, kp1_ref,
                   cls_ref, cnt_ref, tsum_ref, ssum_ref, tcnt_ref, scnt_ref,
                   topk_ref, tgt_ref, blab_ref,
                   l1_ref, l2_ref):
    x = x_ref[...]
    B = x.shape[0]
    eyeb = (jax.lax.broadcasted_iota(jnp.int32, (B, B), 0)
            == jax.lax.broadcasted_iota(jnp.int32, (B, B), 1)).astype(
        jnp.float32)

    def to_row(vcol):
        return jnp.sum(vcol * eyeb, axis=0, keepdims=True)

    sim = jnp.dot(cls_ref[...], x.T,
                  preferred_element_type=jnp.float32) / TEMP
    counts = cnt_ref[...][:, :1]

    l12 = 0.5 * (jnp.sum(x * xs_ref[...], axis=1, keepdims=True)
                 + jnp.sum(x * xs1_ref[...], axis=1, keepdims=True))
    tksum = TEMP * jnp.sum(topk_ref[...][:, :10], axis=1, keepdims=True)
    diag_row = (to_row(l12) + to_row(tksum)) / TEMP

    tgt_row = tgt_ref[...][:1, :].astype(jnp.int32)
    iol = jax.lax.broadcasted_iota(jnp.int32, (L, B), 0)
    oht = (iol == tgt_row).astype(jnp.float32)
    sim = sim + oht * diag_row

    help_pos = jnp.max(oht, axis=1, keepdims=True)
    kp1 = kp1_ref[0, 0]
    nums = counts + help_pos * kp1
    denom = jnp.where(nums > 0, nums, 1.0)
    sim = sim / denom
    maskf = (nums > 0).astype(jnp.float32)
    exps = jnp.exp(sim) * maskf
    sums = jnp.sum(exps, axis=0, keepdims=True) + 1e-6
    p = exps / sums
    p_t = jnp.sum(oht * p, axis=0, keepdims=True)
    l1_ref[...] = -(jnp.sum(jnp.log(p_t + 1e-6), axis=1, keepdims=True)
                    / float(B))

    blab_col = blab_ref[...][:, :1]
    cir = jax.lax.broadcasted_iota(jnp.int32, (B, CLS), 1)
    ohb = (blab_col.astype(jnp.int32) == cir).astype(jnp.float32)
    selt = jnp.dot(ohb, tsum_ref[...], preferred_element_type=jnp.float32)
    sels = jnp.dot(ohb, ssum_ref[...], preferred_element_type=jnp.float32)
    tn = jnp.dot(ohb, tcnt_ref[...][:, :1],
                 preferred_element_type=jnp.float32)
    sn = jnp.dot(ohb, scnt_ref[...][:, :1],
                 preferred_element_type=jnp.float32)
    ad_t = (jnp.sum(selt * x, axis=1, keepdims=True) / TEMP
            / jnp.where(tn > 0, tn, 1.0))
    ad_s = (jnp.sum(sels * x, axis=1, keepdims=True) / TEMP
            / jnp.where(sn > 0, sn, 1.0))
    help_row = 2.0 * (tgt_row >= CLS).astype(jnp.float32) - 1.0
    l2_ref[...] = (jnp.sum(help_row * to_row(ad_t - ad_s), axis=1,
                           keepdims=True) / float(B))


def kernel(inputs, inputs_s, inputs_s1, indexes, k, features, labels,
           pred_labels):
    B, d = inputs.shape
    M = features.shape[0]
    TILE = 1024
    G = -(-M // TILE)
    Mp = G * TILE

    f_p = jnp.pad(features, ((0, Mp - M), (0, 0)))
    lab = jnp.pad(labels.astype(jnp.int32), (0, Mp - M), constant_values=-1)
    plab = jnp.pad(pred_labels.astype(jnp.int32), (0, Mp - M),
                   constant_values=-1)
    labr = lab.reshape(G, 1, TILE)
    labc = lab.reshape(G * TILE, 1)
    plabr = plab.reshape(G, 1, TILE)
    idx = indexes.astype(jnp.int32)
    idxr = idx.reshape(1, B)
    idxc = idx.reshape(B, 1)

    body = functools.partial(_bank_kernel, M, TILE)
    outs = pl.pallas_call(
        body,
        grid=(G,),
        in_specs=[
            pl.BlockSpec((B, d), lambda i: (0, 0)),
            pl.BlockSpec((TILE, d), lambda i: (i, 0)),
            pl.BlockSpec((1, 1, TILE), lambda i: (i, 0, 0)),
            pl.BlockSpec((TILE, 1), lambda i: (i, 0)),
            pl.BlockSpec((1, 1, TILE), lambda i: (i, 0, 0)),
            pl.BlockSpec((1, B), lambda i: (0, 0)),
            pl.BlockSpec((B, 1), lambda i: (0, 0)),
        ],
        out_specs=[
            pl.BlockSpec((L, d), lambda i: (0, 0)),
            pl.BlockSpec((L, d), lambda i: (0, 0)),
            pl.BlockSpec((CLS, d), lambda i: (0, 0)),
            pl.BlockSpec((CLS, d), lambda i: (0, 0)),
            pl.BlockSpec((CLS, d), lambda i: (0, 0)),
            pl.BlockSpec((CLS, d), lambda i: (0, 0)),
            pl.BlockSpec((B, 128), lambda i: (0, 0)),
            pl.BlockSpec((1, B), lambda i: (0, 0)),
            pl.BlockSpec((B, 1), lambda i: (0, 0)),
        ],
        out_shape=[
            jax.ShapeDtypeStruct((L, d), jnp.float32),
            jax.ShapeDtypeStruct((L, d), jnp.float32),
            jax.ShapeDtypeStruct((CLS, d), jnp.float32),
            jax.ShapeDtypeStruct((CLS, d), jnp.float32),
            jax.ShapeDtypeStruct((CLS, d), jnp.float32),
            jax.ShapeDtypeStruct((CLS, d), jnp.float32),
            jax.ShapeDtypeStruct((B, 128), jnp.float32),
            jax.ShapeDtypeStruct((1, B), jnp.float32),
            jax.ShapeDtypeStruct((B, 1), jnp.float32),
        ],
    )(inputs, f_p, labr, labc, plabr, idxr, idxc)
    (cls_sum, cls_cnt, t_sum, s_sum, t_cnt, s_cnt,
     topk_buf, tgt_row, blab_col) = outs

    kp1 = jnp.asarray(k, jnp.float32).reshape(1, 1) + 1.0
    l1, l2 = pl.pallas_call(
        _finish_kernel,
        out_shape=[
            jax.ShapeDtypeStruct((1, 1), jnp.float32),
            jax.ShapeDtypeStruct((1, 1), jnp.float32),
        ],
    )(inputs, inputs_s, inputs_s1, kp1,
      cls_sum, cls_cnt, t_sum, s_sum, t_cnt, s_cnt,
      topk_buf, tgt_row, blab_col)
    return (l1[0, 0], l2[0, 0])


# speedup vs baseline: 3.6104x; 1.2222x over previous
import functools

import jax
import jax.numpy as jnp
from jax.experimental import pallas as pl

TEMP = 0.05
CLS = 1000
L = 2 * CLS
NEG = -1e30


def _bank_kernel(m_total, tile,
                 x_ref, f_ref, labr_ref, labc_ref, plabr_ref,
                 idxr_ref, idxc_ref,
                 cls_ref, cnt_ref, tsum_ref, ssum_ref, tcnt_ref, scnt_ref,
                 topk_ref, tgt_ref, blab_ref):
    i = pl.program_id(0)
    first = (i == 0)
    x = x_ref[...]
    f = f_ref[...]
    labr = labr_ref[0]
    labc = labc_ref[...]
    plabr = plabr_ref[0]
    B = x.shape[0]
    T = f.shape[0]

    il = jax.lax.broadcasted_iota(jnp.int32, (L, T), 0)
    oh = (il == labr).astype(jnp.float32)
    csum = jnp.dot(oh, f, preferred_element_type=jnp.float32)
    ccnt = jnp.sum(oh, axis=1, keepdims=True)
    cls_ref[...] = jnp.where(first, 0.0, cls_ref[...]) + csum
    cnt_ref[...] = (jnp.where(first, 0.0, cnt_ref[...])
                    + jnp.broadcast_to(ccnt, cnt_ref.shape))

    ic = jax.lax.broadcasted_iota(jnp.int32, (CLS, T), 0)
    pon = (ic == plabr).astype(jnp.float32)
    tmask = (labr >= CLS).astype(jnp.float32)
    smask = jnp.logical_and(labr >= 0, labr < CLS).astype(jnp.float32)
    toh = pon * tmask
    soh = pon * smask
    tsum_ref[...] = (jnp.where(first, 0.0, tsum_ref[...])
                     + jnp.dot(toh, f, preferred_element_type=jnp.float32))
    ssum_ref[...] = (jnp.where(first, 0.0, ssum_ref[...])
                     + jnp.dot(soh, f, preferred_element_type=jnp.float32))
    tcnt_ref[...] = (jnp.where(first, 0.0, tcnt_ref[...])
                     + jnp.broadcast_to(jnp.sum(toh, axis=1, keepdims=True),
                                        tcnt_ref.shape))
    scnt_ref[...] = (jnp.where(first, 0.0, scnt_ref[...])
                     + jnp.broadcast_to(jnp.sum(soh, axis=1, keepdims=True),
                                        scnt_ref.shape))

    sim = jnp.dot(x, f.T, preferred_element_type=jnp.float32) / TEMP
    gir = jax.lax.broadcasted_iota(jnp.int32, (1, T), 1) + i * T
    sim = jnp.where(gir < m_total, sim, NEG)

    work = sim
    ms = []
    big = jnp.int32(2 ** 30)
    for _ in range(10):
        m = jnp.max(work, axis=1, keepdims=True)
        hit = jnp.where(work == m, gir, big)
        fidx = jnp.min(hit, axis=1, keepdims=True)
        ms.append(m)
        work = jnp.where(gir == fidx, NEG, work)
    tile_top = jnp.concatenate(ms, axis=1)
    cur = jnp.where(first, NEG, topk_ref[...])[:, :10]
    cand = jnp.concatenate([cur, tile_top], axis=1)
    i20 = jax.lax.broadcasted_iota(jnp.int32, (1, 20), 1)
    ms2 = []
    for _ in range(10):
        m = jnp.max(cand, axis=1, keepdims=True)
        hit = jnp.where(cand == m, i20, big)
        fidx = jnp.min(hit, axis=1, keepdims=True)
        ms2.append(m)
        cand = jnp.where(i20 == fidx, NEG, cand)
    newtop = jnp.concatenate(ms2, axis=1)
    topk_ref[...] = jnp.concatenate(
        [newtop, jnp.full((B, topk_ref.shape[1] - 10), NEG, jnp.float32)],
        axis=1)

    idxr = idxr_ref[...]
    idxc = idxc_ref[...]
    gic = jax.lax.broadcasted_iota(jnp.int32, (T, 1), 0) + i * T
    match_tb = (gic == idxr)
    tgt_c = jnp.sum(jnp.where(match_tb, labc.astype(jnp.float32), 0.0),
                    axis=0, keepdims=True)
    tgt_ref[...] = jnp.where(first, 0.0, tgt_ref[...]) + tgt_c
    match_bt = (idxc == gir)
    blab_c = jnp.sum(jnp.where(match_bt, plabr.astype(jnp.float32), 0.0),
                     axis=1, keepdims=True)
    blab_ref[...] = jnp.where(first, 0.0, blab_ref[...]) + blab_c


def _finish_kernel(x_ref, xs_ref, xs1_ref, kp1_ref,
                   cls_ref, cnt_ref, tsum_ref, ssum_ref, tcnt_ref, scnt_ref,
                   topk_ref, tgt_ref, blab_ref,
                   l1_ref, l2_ref):
    x = x_ref[...]
    B = x.shape[0]
    eyeb = (jax.lax.broadcasted_iota(jnp.int32, (B, B), 0)
            == jax.lax.broadcasted_iota(jnp.int32, (B, B), 1)).astype(
        jnp.float32)

    def to_row(vcol):
        return jnp.sum(vcol * eyeb, axis=0, keepdims=True)

    sim = jnp.dot(cls_ref[...], x.T,
                  preferred_element_type=jnp.float32) / TEMP
    counts = cnt_ref[...][:, :1]

    l12 = 0.5 * (jnp.sum(x * xs_ref[...], axis=1, keepdims=True)
                 + jnp.sum(x * xs1_ref[...], axis=1, keepdims=True))
    tksum = TEMP * jnp.sum(topk_ref[...][:, :10], axis=1, keepdims=True)
    diag_row = (to_row(l12) + to_row(tksum)) / TEMP

    tgt_row = tgt_ref[...][:1, :].astype(jnp.int32)
    iol = jax.lax.broadcasted_iota(jnp.int32, (L, B), 0)
    oht = (iol == tgt_row).astype(jnp.float32)
    sim = sim + oht * diag_row

    help_pos = jnp.max(oht, axis=1, keepdims=True)
    kp1 = kp1_ref[0, 0]
    nums = counts + help_pos * kp1
    denom = jnp.where(nums > 0, nums, 1.0)
    sim = sim / denom
    maskf = (nums > 0).astype(jnp.float32)
    exps = jnp.exp(sim) * maskf
    sums = jnp.sum(exps, axis=0, keepdims=True) + 1e-6
    p = exps / sums
    p_t = jnp.sum(oht * p, axis=0, keepdims=True)
    l1_ref[...] = -(jnp.sum(jnp.log(p_t + 1e-6), axis=1, keepdims=True)
                    / float(B))

    blab_col = blab_ref[...][:, :1]
    cir = jax.lax.broadcasted_iota(jnp.int32, (B, CLS), 1)
    ohb = (blab_col.astype(jnp.int32) == cir).astype(jnp.float32)
    selt = jnp.dot(ohb, tsum_ref[...], preferred_element_type=jnp.float32)
    sels = jnp.dot(ohb, ssum_ref[...], preferred_element_type=jnp.float32)
    tn = jnp.dot(ohb, tcnt_ref[...][:, :1],
                 preferred_element_type=jnp.float32)
    sn = jnp.dot(ohb, scnt_ref[...][:, :1],
                 preferred_element_type=jnp.float32)
    ad_t = (jnp.sum(selt * x, axis=1, keepdims=True) / TEMP
            / jnp.where(tn > 0, tn, 1.0))
    ad_s = (jnp.sum(sels * x, axis=1, keepdims=True) / TEMP
            / jnp.where(sn > 0, sn, 1.0))
    help_row = 2.0 * (tgt_row >= CLS).astype(jnp.float32) - 1.0
    l2_ref[...] = (jnp.sum(help_row * to_row(ad_t - ad_s), axis=1,
                           keepdims=True) / float(B))


def kernel(inputs, inputs_s, inputs_s1, indexes, k, features, labels,
           pred_labels):
    B, d = inputs.shape
    M = features.shape[0]
    TILE = 2048
    G = -(-M // TILE)
    Mp = G * TILE

    f_p = jnp.pad(features, ((0, Mp - M), (0, 0)))
    lab = jnp.pad(labels.astype(jnp.int32), (0, Mp - M), constant_values=-1)
    plab = jnp.pad(pred_labels.astype(jnp.int32), (0, Mp - M),
                   constant_values=-1)
    labr = lab.reshape(G, 1, TILE)
    labc = lab.reshape(G * TILE, 1)
    plabr = plab.reshape(G, 1, TILE)
    idx = indexes.astype(jnp.int32)
    idxr = idx.reshape(1, B)
    idxc = idx.reshape(B, 1)

    body = functools.partial(_bank_kernel, M, TILE)
    outs = pl.pallas_call(
        body,
        grid=(G,),
        in_specs=[
            pl.BlockSpec((B, d), lambda i: (0, 0)),
            pl.BlockSpec((TILE, d), lambda i: (i, 0)),
            pl.BlockSpec((1, 1, TILE), lambda i: (i, 0, 0)),
            pl.BlockSpec((TILE, 1), lambda i: (i, 0)),
            pl.BlockSpec((1, 1, TILE), lambda i: (i, 0, 0)),
            pl.BlockSpec((1, B), lambda i: (0, 0)),
            pl.BlockSpec((B, 1), lambda i: (0, 0)),
        ],
        out_specs=[
            pl.BlockSpec((L, d), lambda i: (0, 0)),
            pl.BlockSpec((L, d), lambda i: (0, 0)),
            pl.BlockSpec((CLS, d), lambda i: (0, 0)),
            pl.BlockSpec((CLS, d), lambda i: (0, 0)),
            pl.BlockSpec((CLS, d), lambda i: (0, 0)),
            pl.BlockSpec((CLS, d), lambda i: (0, 0)),
            pl.BlockSpec((B, 128), lambda i: (0, 0)),
            pl.BlockSpec((1, B), lambda i: (0, 0)),
            pl.BlockSpec((B, 1), lambda i: (0, 0)),
        ],
        out_shape=[
            jax.ShapeDtypeStruct((L, d), jnp.float32),
            jax.ShapeDtypeStruct((L, d), jnp.float32),
            jax.ShapeDtypeStruct((CLS, d), jnp.float32),
            jax.ShapeDtypeStruct((CLS, d), jnp.float32),
            jax.ShapeDtypeStruct((CLS, d), jnp.float32),
            jax.ShapeDtypeStruct((CLS, d), jnp.float32),
            jax.ShapeDtypeStruct((B, 128), jnp.float32),
            jax.ShapeDtypeStruct((1, B), jnp.float32),
            jax.ShapeDtypeStruct((B, 1), jnp.float32),
        ],
    )(inputs, f_p, labr, labc, plabr, idxr, idxc)
    (cls_sum, cls_cnt, t_sum, s_sum, t_cnt, s_cnt,
     topk_buf, tgt_row, blab_col) = outs

    kp1 = jnp.asarray(k, jnp.float32).reshape(1, 1) + 1.0
    l1, l2 = pl.pallas_call(
        _finish_kernel,
        out_shape=[
            jax.ShapeDtypeStruct((1, 1), jnp.float32),
            jax.ShapeDtypeStruct((1, 1), jnp.float32),
        ],
    )(inputs, inputs_s, inputs_s1, kp1,
      cls_sum, cls_cnt, t_sum, s_sum, t_cnt, s_cnt,
      topk_buf, tgt_row, blab_col)
    return (l1[0, 0], l2[0, 0])
